# baseline (speedup 1.0000x reference)
import functools

import jax
import jax.numpy as jnp
from jax import lax
from jax.experimental import pallas as pl
from jax.experimental.pallas import tpu as pltpu
from jax.experimental.pallas import tpu_sc as plsc

B = 4096
NEG = 16
D = 64
N = 1000000
MARGIN = 24.0

NC = 2
NS = 16
L = 16
NW = NC * NS
PP = B // NW
PN = (B * NEG) // NW
C = 128
NCH = PN // C

TBLK = 16384
HALF = TBLK // 2
QSH = 14
TGRID = (N + TBLK - 1) // TBLK
NPAIR = TGRID * HALF


def _transpose_body(xlo_ref, xhi_ref, o_ref):
    eye = jnp.eye(D, dtype=jnp.float32)
    dn = (((0,), (0,)), ((), ()))
    o_ref[:, 0:64] = lax.dot_general(xlo_ref[...], eye, dn,
                                     preferred_element_type=jnp.float32)
    o_ref[:, 64:128] = lax.dot_general(xhi_ref[...], eye, dn,
                                       preferred_element_type=jnp.float32)


def _pair_table(xT):
    return pl.pallas_call(
        _transpose_body,
        grid=(TGRID,),
        in_specs=[pl.BlockSpec((D, HALF), lambda i: (0, 2 * i)),
                  pl.BlockSpec((D, HALF),
                               lambda i: (0, jnp.minimum(2 * i + 1,
                                                         (N - 1) // HALF)))],
        out_specs=pl.BlockSpec((HALF, 128), lambda i: (i, 0)),
        out_shape=jax.ShapeDtypeStruct((NPAIR, 128), jnp.float32),
    )(xT, xT)


def _sc_body(ent_hbm, rel_hbm, hp_hbm, rp_hbm, tp_hbm, hn_hbm, rn_hbm, tn_hbm,
             w_hbm, posd_hbm, posdot_hbm, negd_hbm,
             idx_h, idx_r, idx_t, qh, qr, qt, lbh, lbr, lbt,
             hrows, rrows, trows, wv, posd_v, posdot_v, negd_v, sem):
    wid = lax.axis_index("c") * NS + lax.axis_index("s")

    pltpu.sync_copy(w_hbm, wv)

    def stage(h_src, r_src, t_src, base, par):
        pltpu.sync_copy(h_src.at[pl.ds(base, C)], idx_h)
        pltpu.sync_copy(r_src.at[pl.ds(base, C)], idx_r)
        pltpu.sync_copy(t_src.at[pl.ds(base, C)], idx_t)

        def conv_all(g, _):
            sl = pl.ds(g * L, L)
            ivh = idx_h[sl]
            qh[par, sl] = ((ivh >> 14) << 13) + (ivh & 8191)
            lbh[par, sl] = ((ivh >> 13) & 1) << 6
            ivr = idx_r[sl]
            qr[par, sl] = ((ivr >> 14) << 13) + (ivr & 8191)
            lbr[par, sl] = ((ivr >> 13) & 1) << 6
            ivt = idx_t[sl]
            qt[par, sl] = ((ivt >> 14) << 13) + (ivt & 8191)
            lbt[par, sl] = ((ivt >> 13) & 1) << 6
            return _

        lax.fori_loop(0, C // L, conv_all, 0)

    def fire(par):
        dsl = pl.ds(par * C, C)
        c1 = pltpu.async_copy(ent_hbm.at[qh.at[par]], hrows.at[dsl], sem)
        c2 = pltpu.async_copy(rel_hbm.at[qr.at[par]], rrows.at[dsl], sem)
        c3 = pltpu.async_copy(ent_hbm.at[qt.at[par]], trows.at[dsl], sem)
        return (c1, c2, c3)

    stage(hp_hbm, rp_hbm, tp_hbm, wid * PP, 0)
    for cp in fire(0):
        cp.wait()
    wregs = [wv[pl.ds(k * L, L)] for k in range(D // L)]

    def pos_group(g, _):
        sl = pl.ds(g * L, L)
        rowv = g * L + lax.broadcasted_iota(jnp.int32, (L,), 0)
        lh = lbh[0, sl]
        lr = lbr[0, sl]
        lt = lbt[0, sl]
        acc = jnp.zeros((L,), jnp.float32)
        dot = jnp.zeros((L,), jnp.float32)
        for d in range(D):
            hv = plsc.load_gather(hrows, [rowv, lh + d])
            rv = plsc.load_gather(rrows, [rowv, lr + d])
            tv = plsc.load_gather(trows, [rowv, lt + d])
            s = hv + rv - tv
            acc = acc + jnp.abs(s)
            wd = jnp.take(wregs[d // L], jnp.full((L,), d % L, jnp.int32))
            dot = dot + s * wd
        posd_v[sl] = acc
        posdot_v[sl] = dot
        return _

    lax.fori_loop(0, PP // L, pos_group, 0)

    def neg_compute(c, par):
        def group(g, _):
            sl = pl.ds(g * L, L)
            rowv = par * C + g * L + lax.broadcasted_iota(jnp.int32, (L,), 0)
            lh = lbh[par, sl]
            lr = lbr[par, sl]
            lt = lbt[par, sl]

            def dim4(dg, carry):
                acc, lvh, lvr, lvt = carry
                for _k in range(4):
                    hv = plsc.load_gather(hrows, [rowv, lvh])
                    rv = plsc.load_gather(rrows, [rowv, lvr])
                    tv = plsc.load_gather(trows, [rowv, lvt])
                    acc = acc + jnp.abs(hv + rv - tv)
                    lvh = lvh + 1
                    lvr = lvr + 1
                    lvt = lvt + 1
                return (acc, lvh, lvr, lvt)

            acc, _1, _2, _3 = lax.fori_loop(
                0, D // 4, dim4,
                (jnp.zeros((L,), jnp.float32), lh, lr, lt))
            negd_v[pl.ds(c * C + g * L, L)] = acc
            return _

        lax.fori_loop(0, C // L, group, 0)

    stage(hn_hbm, rn_hbm, tn_hbm, wid * PN, 0)
    inflight = fire(0)
    for c in range(NCH):
        par = c % 2
        nxt = (c + 1) % 2
        if c + 1 < NCH:
            stage(hn_hbm, rn_hbm, tn_hbm, wid * PN + (c + 1) * C, nxt)
            for cp in inflight:
                cp.wait()
            nxt_inflight = fire(nxt)
        else:
            for cp in inflight:
                cp.wait()
            nxt_inflight = None
        neg_compute(c, par)
        inflight = nxt_inflight

    pltpu.sync_copy(posd_v, posd_hbm.at[pl.ds(wid * PP, PP)])
    pltpu.sync_copy(posdot_v, posdot_hbm.at[pl.ds(wid * PP, PP)])
    pltpu.sync_copy(negd_v, negd_hbm.at[pl.ds(wid * PN, PN)])


_sc_call = pl.kernel(
    _sc_body,
    out_type=[
        jax.ShapeDtypeStruct((B,), jnp.float32),
        jax.ShapeDtypeStruct((B,), jnp.float32),
        jax.ShapeDtypeStruct((B * NEG,), jnp.float32),
    ],
    mesh=plsc.VectorSubcoreMesh(core_axis_name="c", subcore_axis_name="s",
                                num_cores=NC, num_subcores=NS),
    compiler_params=pltpu.CompilerParams(needs_layout_passes=False),
    scratch_types=[
        pltpu.VMEM((C,), jnp.int32),
        pltpu.VMEM((C,), jnp.int32),
        pltpu.VMEM((C,), jnp.int32),
        pltpu.VMEM((2, C), jnp.int32),
        pltpu.VMEM((2, C), jnp.int32),
        pltpu.VMEM((2, C), jnp.int32),
        pltpu.VMEM((2, C), jnp.int32),
        pltpu.VMEM((2, C), jnp.int32),
        pltpu.VMEM((2, C), jnp.int32),
        pltpu.VMEM((2 * C, 128), jnp.float32),
        pltpu.VMEM((2 * C, 128), jnp.float32),
        pltpu.VMEM((2 * C, 128), jnp.float32),
        pltpu.VMEM((D,), jnp.float32),
        pltpu.VMEM((PP,), jnp.float32),
        pltpu.VMEM((PP,), jnp.float32),
        pltpu.VMEM((PN,), jnp.float32),
        pltpu.SemaphoreType.DMA,
    ],
)


def _combine_body(pd_ref, dot_ref, nd_ref, db_ref, o_ref):
    db = db_ref[0, 0]
    disc = jax.nn.sigmoid(dot_ref[...] + db)
    pos = -jnp.log(jax.nn.sigmoid(MARGIN - pd_ref[...]))
    neg = jnp.sum((1.0 / NEG) * jnp.log(jax.nn.sigmoid(MARGIN - nd_ref[...])),
                  axis=1, keepdims=True)
    o_ref[...] = disc * (pos + neg)


def _combine(pd, dot, nd, db):
    return pl.pallas_call(
        _combine_body,
        out_shape=jax.ShapeDtypeStruct((B, 1), jnp.float32),
    )(pd.reshape(B, 1), dot.reshape(B, 1), nd.reshape(B, NEG), db.reshape(1, 1))


def kernel(positive_triples, block_of_negative_triples, negative_sample_size,
           entities_emb, relations_emb, D_W, D_b):
    hp = positive_triples[:, 0]
    rp = positive_triples[:, 1]
    tp = positive_triples[:, 2]
    nflat = block_of_negative_triples.reshape(B * NEG, 3)
    hn = nflat[:, 0]
    rn = nflat[:, 1]
    tn = nflat[:, 2]
    w = D_W.reshape(D)

    ent128 = _pair_table(entities_emb.T)
    rel128 = _pair_table(relations_emb.T)

    posd, posdot, negd = _sc_call(ent128, rel128,
                                  hp, rp, tp, hn, rn, tn, w)
    out = _combine(posd, posdot, negd.reshape(B, NEG), D_b)
    return out.reshape(B)


# speedup vs baseline: 1.0306x; 1.0306x over previous
import functools

import jax
import jax.numpy as jnp
from jax import lax
from jax.experimental import pallas as pl
from jax.experimental.pallas import tpu as pltpu
from jax.experimental.pallas import tpu_sc as plsc

B = 4096
NEG = 16
D = 64
N = 1000000
MARGIN = 24.0

NC = 2
NS = 16
L = 16
NW = NC * NS
PP = B // NW
PN = (B * NEG) // NW
C = 128
NCH = PN // C

TBLK = 16384
HALF = TBLK // 2
QSH = 14
TGRID = (N + TBLK - 1) // TBLK
NPAIR = TGRID * HALF


def _transpose_body(xlo_ref, xhi_ref, o_ref):
    o_ref[:, 0:64] = xlo_ref[...].T
    o_ref[:, 64:128] = xhi_ref[...].T


def _pair_table(xT):
    return pl.pallas_call(
        _transpose_body,
        grid=(TGRID,),
        in_specs=[pl.BlockSpec((D, HALF), lambda i: (0, 2 * i)),
                  pl.BlockSpec((D, HALF),
                               lambda i: (0, jnp.minimum(2 * i + 1,
                                                         (N - 1) // HALF)))],
        out_specs=pl.BlockSpec((HALF, 128), lambda i: (i, 0)),
        out_shape=jax.ShapeDtypeStruct((NPAIR, 128), jnp.float32),
    )(xT, xT)


def _sc_body(ent_hbm, rel_hbm, hp_hbm, rp_hbm, tp_hbm, hn_hbm, rn_hbm, tn_hbm,
             w_hbm, posd_hbm, posdot_hbm, negd_hbm,
             raw_h, raw_r, raw_t, lb_h, lb_r, lb_t,
             hrows, rrows, trows, wv, posd_v, posdot_v, negd_v, sem0, sem1):
    wid = lax.axis_index("c") * NS + lax.axis_index("s")
    NTOT = PP + PN
    NCHK = NTOT // C

    pltpu.sync_copy(w_hbm, wv)
    pltpu.sync_copy(hp_hbm.at[pl.ds(wid * PP, PP)], raw_h.at[pl.ds(0, PP)])
    pltpu.sync_copy(rp_hbm.at[pl.ds(wid * PP, PP)], raw_r.at[pl.ds(0, PP)])
    pltpu.sync_copy(tp_hbm.at[pl.ds(wid * PP, PP)], raw_t.at[pl.ds(0, PP)])
    pltpu.sync_copy(hn_hbm.at[pl.ds(wid * PN, PN)], raw_h.at[pl.ds(PP, PN)])
    pltpu.sync_copy(rn_hbm.at[pl.ds(wid * PN, PN)], raw_r.at[pl.ds(PP, PN)])
    pltpu.sync_copy(tn_hbm.at[pl.ds(wid * PN, PN)], raw_t.at[pl.ds(PP, PN)])

    def conv(g, _):
        sl = pl.ds(g * L, L)
        ivh = raw_h[sl]
        raw_h[sl] = ((ivh >> 14) << 13) + (ivh & 8191)
        lb_h[sl] = ((ivh >> 13) & 1) << 6
        ivr = raw_r[sl]
        raw_r[sl] = ((ivr >> 14) << 13) + (ivr & 8191)
        lb_r[sl] = ((ivr >> 13) & 1) << 6
        ivt = raw_t[sl]
        raw_t[sl] = ((ivt >> 14) << 13) + (ivt & 8191)
        lb_t[sl] = ((ivt >> 13) & 1) << 6
        return _

    lax.fori_loop(0, NTOT // L, conv, 0)

    sems = (sem0, sem1)

    def fire(k):
        par = k % 2
        dsl = pl.ds(par * C, C)
        isl = pl.ds(k * C, C)
        sem = sems[par]
        c1 = pltpu.async_copy(ent_hbm.at[raw_h.at[isl]], hrows.at[dsl], sem)
        c2 = pltpu.async_copy(rel_hbm.at[raw_r.at[isl]], rrows.at[dsl], sem)
        c3 = pltpu.async_copy(ent_hbm.at[raw_t.at[isl]], trows.at[dsl], sem)
        return (c1, c2, c3)

    wregs = [wv[pl.ds(k * L, L)] for k in range(D // L)]

    def pos_compute():
        def pos_group(g, _):
            sl = pl.ds(g * L, L)
            rowv = g * L + lax.broadcasted_iota(jnp.int32, (L,), 0)
            lh = lb_h[sl]
            lr = lb_r[sl]
            lt = lb_t[sl]
            acc = jnp.zeros((L,), jnp.float32)
            dot = jnp.zeros((L,), jnp.float32)
            for d in range(D):
                hv = plsc.load_gather(hrows, [rowv, lh + d])
                rv = plsc.load_gather(rrows, [rowv, lr + d])
                tv = plsc.load_gather(trows, [rowv, lt + d])
                sv = hv + rv - tv
                acc = acc + jnp.abs(sv)
                wd = jnp.take(wregs[d // L], jnp.full((L,), d % L, jnp.int32))
                dot = dot + sv * wd
            posd_v[sl] = acc
            posdot_v[sl] = dot
            return _

        lax.fori_loop(0, PP // L, pos_group, 0)

    def neg_compute(k):
        par = k % 2
        cbase = (k - 1) * C

        def group(g, _):
            rowv = par * C + g * L + lax.broadcasted_iota(jnp.int32, (L,), 0)
            lsl = pl.ds(k * C + g * L, L)
            lh = lb_h[lsl]
            lr = lb_r[lsl]
            lt = lb_t[lsl]

            def dim4(dg, carry):
                acc, lvh, lvr, lvt = carry
                for _k in range(4):
                    hv = plsc.load_gather(hrows, [rowv, lvh])
                    rv = plsc.load_gather(rrows, [rowv, lvr])
                    tv = plsc.load_gather(trows, [rowv, lvt])
                    acc = acc + jnp.abs(hv + rv - tv)
                    lvh = lvh + 1
                    lvr = lvr + 1
                    lvt = lvt + 1
                return (acc, lvh, lvr, lvt)

            acc, _1, _2, _3 = lax.fori_loop(
                0, D // 4, dim4,
                (jnp.zeros((L,), jnp.float32), lh, lr, lt))
            negd_v[pl.ds(cbase + g * L, L)] = acc
            return _

        lax.fori_loop(0, C // L, group, 0)

    desc = {0: fire(0), 1: fire(1)}
    for k in range(NCHK):
        for cp in desc.pop(k):
            cp.wait()
        if k == 0:
            pos_compute()
        else:
            neg_compute(k)
        if k + 2 < NCHK:
            desc[k + 2] = fire(k + 2)

    pltpu.sync_copy(posd_v, posd_hbm.at[pl.ds(wid * PP, PP)])
    pltpu.sync_copy(posdot_v, posdot_hbm.at[pl.ds(wid * PP, PP)])
    pltpu.sync_copy(negd_v, negd_hbm.at[pl.ds(wid * PN, PN)])


_sc_call = pl.kernel(
    _sc_body,
    out_type=[
        jax.ShapeDtypeStruct((B,), jnp.float32),
        jax.ShapeDtypeStruct((B,), jnp.float32),
        jax.ShapeDtypeStruct((B * NEG,), jnp.float32),
    ],
    mesh=plsc.VectorSubcoreMesh(core_axis_name="c", subcore_axis_name="s",
                                num_cores=NC, num_subcores=NS),
    compiler_params=pltpu.CompilerParams(needs_layout_passes=False),
    scratch_types=[
        pltpu.VMEM((PP + PN,), jnp.int32),
        pltpu.VMEM((PP + PN,), jnp.int32),
        pltpu.VMEM((PP + PN,), jnp.int32),
        pltpu.VMEM((PP + PN,), jnp.int32),
        pltpu.VMEM((PP + PN,), jnp.int32),
        pltpu.VMEM((PP + PN,), jnp.int32),
        pltpu.VMEM((2 * C, 128), jnp.float32),
        pltpu.VMEM((2 * C, 128), jnp.float32),
        pltpu.VMEM((2 * C, 128), jnp.float32),
        pltpu.VMEM((D,), jnp.float32),
        pltpu.VMEM((PP,), jnp.float32),
        pltpu.VMEM((PP,), jnp.float32),
        pltpu.VMEM((PN,), jnp.float32),
        pltpu.SemaphoreType.DMA,
        pltpu.SemaphoreType.DMA,
    ],
)


def _combine_body(pd_ref, dot_ref, nd_ref, db_ref, o_ref):
    db = db_ref[0, 0]
    disc = jax.nn.sigmoid(dot_ref[...] + db)
    pos = -jnp.log(jax.nn.sigmoid(MARGIN - pd_ref[...]))
    neg = jnp.sum((1.0 / NEG) * jnp.log(jax.nn.sigmoid(MARGIN - nd_ref[...])),
                  axis=1, keepdims=True)
    o_ref[...] = disc * (pos + neg)


def _combine(pd, dot, nd, db):
    return pl.pallas_call(
        _combine_body,
        out_shape=jax.ShapeDtypeStruct((B, 1), jnp.float32),
    )(pd.reshape(B, 1), dot.reshape(B, 1), nd.reshape(B, NEG), db.reshape(1, 1))


def kernel(positive_triples, block_of_negative_triples, negative_sample_size,
           entities_emb, relations_emb, D_W, D_b):
    hp = positive_triples[:, 0]
    rp = positive_triples[:, 1]
    tp = positive_triples[:, 2]
    nflat = block_of_negative_triples.reshape(B * NEG, 3)
    hn = nflat[:, 0]
    rn = nflat[:, 1]
    tn = nflat[:, 2]
    w = D_W.reshape(D)

    ent128 = _pair_table(entities_emb.T)
    rel128 = _pair_table(relations_emb.T)

    posd, posdot, negd = _sc_call(ent128, rel128,
                                  hp, rp, tp, hn, rn, tn, w)
    out = _combine(posd, posdot, negd.reshape(B, NEG), D_b)
    return out.reshape(B)


# speedup vs baseline: 1.0755x; 1.0435x over previous
import functools

import jax
import jax.numpy as jnp
from jax import lax
from jax.experimental import pallas as pl
from jax.experimental.pallas import tpu as pltpu
from jax.experimental.pallas import tpu_sc as plsc

B = 4096
NEG = 16
D = 64
N = 1000000
MARGIN = 24.0

NC = 2
NS = 16
L = 16
NW = NC * NS
PP = B // NW
PN = (B * NEG) // NW
C = 128
NCH = PN // C

TBLK = 32768
HALF = TBLK // 2
QSH = 15
TGRID = (N + TBLK - 1) // TBLK
NPAIR = TGRID * HALF


def _transpose_body(xlo_ref, xhi_ref, o_ref):
    o_ref[:, 0:64] = xlo_ref[...].T
    o_ref[:, 64:128] = xhi_ref[...].T


def _pair_table(xT):
    return pl.pallas_call(
        _transpose_body,
        grid=(TGRID,),
        in_specs=[pl.BlockSpec((D, HALF), lambda i: (0, 2 * i)),
                  pl.BlockSpec((D, HALF),
                               lambda i: (0, jnp.minimum(2 * i + 1,
                                                         (N - 1) // HALF)))],
        out_specs=pl.BlockSpec((HALF, 128), lambda i: (i, 0)),
        out_shape=jax.ShapeDtypeStruct((NPAIR, 128), jnp.float32),
    )(xT, xT)


def _sc_body(ent_hbm, rel_hbm, hp_hbm, rp_hbm, tp_hbm, hn_hbm, rn_hbm, tn_hbm,
             w_hbm, posd_hbm, posdot_hbm, negd_hbm,
             raw_h, raw_r, raw_t, lb_h, lb_r, lb_t,
             hrows, rrows, trows, wv, posd_v, posdot_v, negd_v, sem0, sem1):
    wid = lax.axis_index("c") * NS + lax.axis_index("s")
    NTOT = PP + PN
    NCHK = NTOT // C

    pltpu.sync_copy(w_hbm, wv)
    pltpu.sync_copy(hp_hbm.at[pl.ds(wid * PP, PP)], raw_h.at[pl.ds(0, PP)])
    pltpu.sync_copy(rp_hbm.at[pl.ds(wid * PP, PP)], raw_r.at[pl.ds(0, PP)])
    pltpu.sync_copy(tp_hbm.at[pl.ds(wid * PP, PP)], raw_t.at[pl.ds(0, PP)])
    pltpu.sync_copy(hn_hbm.at[pl.ds(wid * PN, PN)], raw_h.at[pl.ds(PP, PN)])
    pltpu.sync_copy(rn_hbm.at[pl.ds(wid * PN, PN)], raw_r.at[pl.ds(PP, PN)])
    pltpu.sync_copy(tn_hbm.at[pl.ds(wid * PN, PN)], raw_t.at[pl.ds(PP, PN)])

    def conv(g, _):
        sl = pl.ds(g * L, L)
        ivh = raw_h[sl]
        raw_h[sl] = ((ivh >> 15) << 14) + (ivh & 16383)
        lb_h[sl] = ((ivh >> 14) & 1) << 6
        ivr = raw_r[sl]
        raw_r[sl] = ((ivr >> 15) << 14) + (ivr & 16383)
        lb_r[sl] = ((ivr >> 14) & 1) << 6
        ivt = raw_t[sl]
        raw_t[sl] = ((ivt >> 15) << 14) + (ivt & 16383)
        lb_t[sl] = ((ivt >> 14) & 1) << 6
        return _

    lax.fori_loop(0, NTOT // L, conv, 0)

    sems = (sem0, sem1)

    def fire(k):
        par = k % 2
        dsl = pl.ds(par * C, C)
        isl = pl.ds(k * C, C)
        sem = sems[par]
        c1 = pltpu.async_copy(ent_hbm.at[raw_h.at[isl]], hrows.at[dsl], sem)
        c2 = pltpu.async_copy(rel_hbm.at[raw_r.at[isl]], rrows.at[dsl], sem)
        c3 = pltpu.async_copy(ent_hbm.at[raw_t.at[isl]], trows.at[dsl], sem)
        return (c1, c2, c3)

    wregs = [wv[pl.ds(k * L, L)] for k in range(D // L)]

    def pos_compute():
        def pos_group(g, _):
            sl = pl.ds(g * L, L)
            rowv = g * L + lax.broadcasted_iota(jnp.int32, (L,), 0)
            lh = lb_h[sl]
            lr = lb_r[sl]
            lt = lb_t[sl]
            acc = jnp.zeros((L,), jnp.float32)
            dot = jnp.zeros((L,), jnp.float32)
            for d in range(D):
                hv = plsc.load_gather(hrows, [rowv, lh + d])
                rv = plsc.load_gather(rrows, [rowv, lr + d])
                tv = plsc.load_gather(trows, [rowv, lt + d])
                sv = hv + rv - tv
                acc = acc + jnp.abs(sv)
                wd = jnp.take(wregs[d // L], jnp.full((L,), d % L, jnp.int32))
                dot = dot + sv * wd
            posd_v[sl] = acc
            posdot_v[sl] = dot
            return _

        lax.fori_loop(0, PP // L, pos_group, 0)

    def neg_compute(k):
        par = k % 2
        cbase = (k - 1) * C

        def group(g, _):
            rowv = par * C + g * L + lax.broadcasted_iota(jnp.int32, (L,), 0)
            lsl = pl.ds(k * C + g * L, L)
            lh = lb_h[lsl]
            lr = lb_r[lsl]
            lt = lb_t[lsl]

            def dim4(dg, carry):
                acc, lvh, lvr, lvt = carry
                for _k in range(4):
                    hv = plsc.load_gather(hrows, [rowv, lvh])
                    rv = plsc.load_gather(rrows, [rowv, lvr])
                    tv = plsc.load_gather(trows, [rowv, lvt])
                    acc = acc + jnp.abs(hv + rv - tv)
                    lvh = lvh + 1
                    lvr = lvr + 1
                    lvt = lvt + 1
                return (acc, lvh, lvr, lvt)

            acc, _1, _2, _3 = lax.fori_loop(
                0, D // 4, dim4,
                (jnp.zeros((L,), jnp.float32), lh, lr, lt))
            negd_v[pl.ds(cbase + g * L, L)] = acc
            return _

        lax.fori_loop(0, C // L, group, 0)

    desc = {0: fire(0), 1: fire(1)}
    for k in range(NCHK):
        for cp in desc.pop(k):
            cp.wait()
        if k == 0:
            pos_compute()
        else:
            neg_compute(k)
        if k + 2 < NCHK:
            desc[k + 2] = fire(k + 2)

    pltpu.sync_copy(posd_v, posd_hbm.at[pl.ds(wid * PP, PP)])
    pltpu.sync_copy(posdot_v, posdot_hbm.at[pl.ds(wid * PP, PP)])
    pltpu.sync_copy(negd_v, negd_hbm.at[pl.ds(wid * PN, PN)])


_sc_call = pl.kernel(
    _sc_body,
    out_type=[
        jax.ShapeDtypeStruct((B,), jnp.float32),
        jax.ShapeDtypeStruct((B,), jnp.float32),
        jax.ShapeDtypeStruct((B * NEG,), jnp.float32),
    ],
    mesh=plsc.VectorSubcoreMesh(core_axis_name="c", subcore_axis_name="s",
                                num_cores=NC, num_subcores=NS),
    compiler_params=pltpu.CompilerParams(needs_layout_passes=False),
    scratch_types=[
        pltpu.VMEM((PP + PN,), jnp.int32),
        pltpu.VMEM((PP + PN,), jnp.int32),
        pltpu.VMEM((PP + PN,), jnp.int32),
        pltpu.VMEM((PP + PN,), jnp.int32),
        pltpu.VMEM((PP + PN,), jnp.int32),
        pltpu.VMEM((PP + PN,), jnp.int32),
        pltpu.VMEM((2 * C, 128), jnp.float32),
        pltpu.VMEM((2 * C, 128), jnp.float32),
        pltpu.VMEM((2 * C, 128), jnp.float32),
        pltpu.VMEM((D,), jnp.float32),
        pltpu.VMEM((PP,), jnp.float32),
        pltpu.VMEM((PP,), jnp.float32),
        pltpu.VMEM((PN,), jnp.float32),
        pltpu.SemaphoreType.DMA,
        pltpu.SemaphoreType.DMA,
    ],
)


def _combine_body(pd_ref, dot_ref, nd_ref, db_ref, o_ref):
    db = db_ref[0, 0]
    disc = jax.nn.sigmoid(dot_ref[...] + db)
    pos = -jnp.log(jax.nn.sigmoid(MARGIN - pd_ref[...]))
    neg = jnp.sum((1.0 / NEG) * jnp.log(jax.nn.sigmoid(MARGIN - nd_ref[...])),
                  axis=1, keepdims=True)
    o_ref[...] = disc * (pos + neg)


def _combine(pd, dot, nd, db):
    return pl.pallas_call(
        _combine_body,
        out_shape=jax.ShapeDtypeStruct((B, 1), jnp.float32),
    )(pd.reshape(B, 1), dot.reshape(B, 1), nd.reshape(B, NEG), db.reshape(1, 1))


def kernel(positive_triples, block_of_negative_triples, negative_sample_size,
           entities_emb, relations_emb, D_W, D_b):
    hp = positive_triples[:, 0]
    rp = positive_triples[:, 1]
    tp = positive_triples[:, 2]
    nflat = block_of_negative_triples.reshape(B * NEG, 3)
    hn = nflat[:, 0]
    rn = nflat[:, 1]
    tn = nflat[:, 2]
    w = D_W.reshape(D)

    ent128 = _pair_table(entities_emb.T)
    rel128 = _pair_table(relations_emb.T)

    posd, posdot, negd = _sc_call(ent128, rel128,
                                  hp, rp, tp, hn, rn, tn, w)
    out = _combine(posd, posdot, negd.reshape(B, NEG), D_b)
    return out.reshape(B)
